# baseline (speedup 1.0000x reference)
import functools

import jax
import jax.numpy as jnp
from jax import lax
from jax.experimental import pallas as pl
from jax.experimental.pallas import tpu as pltpu
from jax.experimental.pallas import tpu_sc as plsc

N_NODES_C = 100000
MAX_DEGREE = 512
EMB_DIM = 16

NC, NS, L = 2, 16, 16
NW = NC * NS

E_TOTAL = 3200000
EP = E_TOTAL // NW
CH = 4000
NCH = EP // CH

NT = 3136
NSEG = NT // L
NPAD = NW * NT
N_LAST = N_NODES_C - (NW - 1) * NT

_mesh = plsc.VectorSubcoreMesh(core_axis_name="c", subcore_axis_name="s")
_params = pltpu.CompilerParams(needs_layout_passes=False)


def _wid():
    return lax.axis_index("s") * NC + lax.axis_index("c")


def _hist_body(src_ref, part_ref, hist, ebuf0, ebuf1, sem0, sem1):
    wid = _wid()
    ebase = wid * EP

    zeros16 = jnp.zeros((L,), jnp.int32)
    ones16 = jnp.ones((L,), jnp.int32)

    def start_chunk(k, buf, sem):
        pltpu.make_async_copy(
            src_ref.at[pl.ds(ebase + k * CH, CH)], buf, sem).start()

    def wait_chunk(buf, sem):
        pltpu.make_async_copy(src_ref.at[pl.ds(0, CH)], buf, sem).wait()

    start_chunk(0, ebuf0, sem0)
    start_chunk(1, ebuf1, sem1)

    @plsc.parallel_loop(0, NPAD // L, 1, unroll=16)
    def _(i):
        hist[pl.ds(i * L, L)] = zeros16

    def do_chunk(buf):
        @plsc.parallel_loop(0, CH // L, 1, unroll=8)
        def _(i):
            plsc.addupdate_scatter(hist, [buf[pl.ds(i * L, L)]], ones16)

    wait_chunk(ebuf0, sem0)
    do_chunk(ebuf0)
    start_chunk(2, ebuf0, sem0)

    def pair_body(j, carry):
        k1 = 2 * j + 1
        wait_chunk(ebuf1, sem1)
        do_chunk(ebuf1)
        @pl.when(k1 + 2 < NCH)
        def _():
            start_chunk(k1 + 2, ebuf1, sem1)
        wait_chunk(ebuf0, sem0)
        do_chunk(ebuf0)
        @pl.when(k1 + 3 < NCH)
        def _():
            start_chunk(k1 + 3, ebuf0, sem0)
        return carry

    lax.fori_loop(0, (NCH - 1) // 2, pair_body, 0)

    pltpu.sync_copy(hist, part_ref.at[pl.ds(wid * NPAD, NPAD)])


_hist_kernel = functools.partial(
    pl.kernel,
    out_type=jax.ShapeDtypeStruct((NW * NPAD,), jnp.int32),
    mesh=_mesh,
    compiler_params=_params,
    scratch_types=[
        pltpu.VMEM((NPAD,), jnp.int32),
        pltpu.VMEM((CH,), jnp.int32),
        pltpu.VMEM((CH,), jnp.int32),
        pltpu.SemaphoreType.DMA,
        pltpu.SemaphoreType.DMA,
    ],
)(_hist_body)


def _lookup_body(part_ref, emb_ref, out_ref, emb_v, acc,
                 pbuf0, pbuf1, pbuf2, pbuf3, outbuf,
                 sem0, sem1, sem2, sem3):
    wid = _wid()
    base = wid * NT
    pbufs = (pbuf0, pbuf1, pbuf2, pbuf3)
    sems = (sem0, sem1, sem2, sem3)

    def start_row(r, buf, sem):
        pltpu.make_async_copy(
            part_ref.at[pl.ds(r * NPAD + base, NT)], buf, sem).start()

    def wait_row(buf, sem):
        pltpu.make_async_copy(part_ref.at[pl.ds(0, NT)], buf, sem).wait()

    def addrow(buf):
        @plsc.parallel_loop(0, NSEG, 1, unroll=8)
        def _(s):
            sl = pl.ds(s * L, L)
            acc[sl] = acc[sl] + buf[sl]

    for b in range(4):
        start_row(1 + b, pbufs[b], sems[b])
    pltpu.sync_copy(emb_ref, emb_v)
    pltpu.sync_copy(part_ref.at[pl.ds(base, NT)], acc)

    def ring_body(j, carry):
        for b in range(4):
            r = 4 * j + 1 + b
            wait_row(pbufs[b], sems[b])
            addrow(pbufs[b])
            @pl.when(r + 4 < NW)
            def _():
                start_row(r + 4, pbufs[b], sems[b])
        return carry

    lax.fori_loop(0, 7, ring_body, 0)
    for b in range(3):
        wait_row(pbufs[b], sems[b])
        addrow(pbufs[b])

    iota16 = lax.iota(jnp.int32, L)
    row16 = iota16 * EMB_DIM

    @plsc.parallel_loop(0, NSEG, 1, unroll=2)
    def _(s):
        d = acc[pl.ds(s * L, L)]
        dc = jnp.minimum(jnp.maximum(d, 1), MAX_DEGREE) - 1
        src_base = dc * EMB_DIM
        dst_base = s * (L * EMB_DIM) + row16
        for c in range(EMB_DIM):
            vals = plsc.load_gather(emb_v, [src_base + c])
            plsc.store_scatter(outbuf, [dst_base + c], vals)

    @pl.when(wid < NW - 1)
    def _():
        pltpu.sync_copy(outbuf, out_ref.at[pl.ds(base * EMB_DIM, NT * EMB_DIM)])

    @pl.when(wid == NW - 1)
    def _():
        pltpu.sync_copy(outbuf.at[pl.ds(0, N_LAST * EMB_DIM)],
                        out_ref.at[pl.ds(base * EMB_DIM, N_LAST * EMB_DIM)])


_lookup_kernel = functools.partial(
    pl.kernel,
    out_type=jax.ShapeDtypeStruct((N_NODES_C * EMB_DIM,), jnp.float32),
    mesh=_mesh,
    compiler_params=_params,
    scratch_types=[
        pltpu.VMEM((MAX_DEGREE * EMB_DIM,), jnp.float32),
        pltpu.VMEM((NT,), jnp.int32),
        pltpu.VMEM((NT,), jnp.int32),
        pltpu.VMEM((NT,), jnp.int32),
        pltpu.VMEM((NT,), jnp.int32),
        pltpu.VMEM((NT,), jnp.int32),
        pltpu.VMEM((NT * EMB_DIM,), jnp.float32),
        pltpu.SemaphoreType.DMA,
        pltpu.SemaphoreType.DMA,
        pltpu.SemaphoreType.DMA,
        pltpu.SemaphoreType.DMA,
    ],
)(_lookup_body)


def kernel(edge_index, num_nodes, emb_weight):
    part = _hist_kernel(edge_index[0])
    out_flat = _lookup_kernel(part, emb_weight.reshape(-1))
    return out_flat.reshape(N_NODES_C, EMB_DIM)


# speedup vs baseline: 1.0579x; 1.0579x over previous
import functools

import jax
import jax.numpy as jnp
from jax import lax
from jax.experimental import pallas as pl
from jax.experimental.pallas import tpu as pltpu
from jax.experimental.pallas import tpu_sc as plsc

N_NODES_C = 100000
MAX_DEGREE = 512
EMB_DIM = 16

NC, NS, L = 2, 16, 16
NW = NC * NS

E_TOTAL = 3200000
EP = E_TOTAL // NW
RW = 100096
CH = 4096
N_FULL = RW // CH
CH_T = RW - N_FULL * CH

NT = 3136
NSEG = NT // L
NPAD = NW * NT
N_LAST = N_NODES_C - (NW - 1) * NT

_mesh = plsc.VectorSubcoreMesh(core_axis_name="c", subcore_axis_name="s")
_params = pltpu.CompilerParams(needs_layout_passes=False)


def _wid():
    return lax.axis_index("s") * NC + lax.axis_index("c")


def _hist_body(edge_ref, part_ref, hist, ebuf0, ebuf1, sem0, sem1):
    wid = _wid()
    wstart = wid * EP
    awin = wstart - lax.rem(wstart, 128)
    lo = wstart - awin

    zeros16 = jnp.zeros((L,), jnp.int32)
    ones16 = jnp.ones((L,), jnp.int32)
    iota16 = lax.iota(jnp.int32, L)

    def start_chunk(k, buf, sem, ncols):
        off = pl.multiple_of(awin + k * CH, 128)
        pltpu.make_async_copy(
            edge_ref.at[:, pl.ds(off, ncols)],
            buf.at[:, pl.ds(0, ncols)], sem).start()

    def wait_chunk(buf, sem, ncols):
        pltpu.make_async_copy(
            edge_ref.at[:, pl.ds(0, ncols)],
            buf.at[:, pl.ds(0, ncols)], sem).wait()

    def do_group(buf, i, mask):
        idx = buf[0, pl.ds(i * L, L)]
        plsc.addupdate_scatter(hist, [idx], ones16, mask=mask)

    def do_chunk(buf):
        @plsc.parallel_loop(0, CH // L, 1, unroll=8)
        def _(i):
            do_group(buf, i, None)

    def do_chunk_masked_lo(buf):
        @plsc.parallel_loop(0, CH // L, 1, unroll=8)
        def _(i):
            mask = (i * L + iota16) >= lo
            do_group(buf, i, mask)

    def do_chunk_masked_hi(buf):
        hi = lo + (EP - N_FULL * CH)
        @plsc.parallel_loop(0, CH_T // L, 1, unroll=8)
        def _(i):
            mask = (i * L + iota16) < hi
            do_group(buf, i, mask)

    start_chunk(0, ebuf0, sem0, CH)
    start_chunk(1, ebuf1, sem1, CH)

    @plsc.parallel_loop(0, NPAD // L, 1, unroll=16)
    def _(i):
        hist[pl.ds(i * L, L)] = zeros16

    wait_chunk(ebuf0, sem0, CH)
    do_chunk_masked_lo(ebuf0)
    start_chunk(2, ebuf0, sem0, CH)

    def pair_body(j, carry):
        k1 = 2 * j + 1
        wait_chunk(ebuf1, sem1, CH)
        do_chunk(ebuf1)
        start_chunk(k1 + 2, ebuf1, sem1, CH)
        wait_chunk(ebuf0, sem0, CH)
        do_chunk(ebuf0)
        @pl.when(k1 + 3 < N_FULL)
        def _():
            start_chunk(k1 + 3, ebuf0, sem0, CH)
        return carry

    lax.fori_loop(0, (N_FULL - 2) // 2, pair_body, 0)
    wait_chunk(ebuf1, sem1, CH)
    do_chunk(ebuf1)
    start_chunk(N_FULL, ebuf0, sem0, CH_T)
    wait_chunk(ebuf0, sem0, CH_T)
    do_chunk_masked_hi(ebuf0)

    pltpu.sync_copy(hist, part_ref.at[pl.ds(wid * NPAD, NPAD)])


_hist_kernel = functools.partial(
    pl.kernel,
    out_type=jax.ShapeDtypeStruct((NW * NPAD,), jnp.int32),
    mesh=_mesh,
    compiler_params=_params,
    scratch_types=[
        pltpu.VMEM((NPAD,), jnp.int32),
        pltpu.VMEM((2, CH), jnp.int32),
        pltpu.VMEM((2, CH), jnp.int32),
        pltpu.SemaphoreType.DMA,
        pltpu.SemaphoreType.DMA,
    ],
)(_hist_body)


def _lookup_body(part_ref, emb_ref, out_ref, emb_v, acc,
                 pbuf0, pbuf1, pbuf2, pbuf3, outbuf,
                 sem0, sem1, sem2, sem3):
    wid = _wid()
    base = wid * NT
    pbufs = (pbuf0, pbuf1, pbuf2, pbuf3)
    sems = (sem0, sem1, sem2, sem3)

    def start_row(r, buf, sem):
        pltpu.make_async_copy(
            part_ref.at[pl.ds(r * NPAD + base, NT)], buf, sem).start()

    def wait_row(buf, sem):
        pltpu.make_async_copy(part_ref.at[pl.ds(0, NT)], buf, sem).wait()

    def addrow(buf):
        @plsc.parallel_loop(0, NSEG, 1, unroll=8)
        def _(s):
            sl = pl.ds(s * L, L)
            acc[sl] = acc[sl] + buf[sl]

    for b in range(4):
        start_row(1 + b, pbufs[b], sems[b])
    pltpu.sync_copy(emb_ref, emb_v)
    pltpu.sync_copy(part_ref.at[pl.ds(base, NT)], acc)

    def ring_body(j, carry):
        for b in range(4):
            r = 4 * j + 1 + b
            wait_row(pbufs[b], sems[b])
            addrow(pbufs[b])
            @pl.when(r + 4 < NW)
            def _():
                start_row(r + 4, pbufs[b], sems[b])
        return carry

    lax.fori_loop(0, 7, ring_body, 0)
    for b in range(3):
        wait_row(pbufs[b], sems[b])
        addrow(pbufs[b])

    iota16 = lax.iota(jnp.int32, L)
    row16 = iota16 * EMB_DIM

    @plsc.parallel_loop(0, NSEG, 1, unroll=2)
    def _(s):
        d = acc[pl.ds(s * L, L)]
        dc = jnp.minimum(jnp.maximum(d, 1), MAX_DEGREE) - 1
        src_base = dc * EMB_DIM
        dst_base = s * (L * EMB_DIM) + row16
        for c in range(EMB_DIM):
            vals = plsc.load_gather(emb_v, [src_base + c])
            plsc.store_scatter(outbuf, [dst_base + c], vals)

    @pl.when(wid < NW - 1)
    def _():
        pltpu.sync_copy(outbuf, out_ref.at[pl.ds(base * EMB_DIM, NT * EMB_DIM)])

    @pl.when(wid == NW - 1)
    def _():
        pltpu.sync_copy(outbuf.at[pl.ds(0, N_LAST * EMB_DIM)],
                        out_ref.at[pl.ds(base * EMB_DIM, N_LAST * EMB_DIM)])


_lookup_kernel = functools.partial(
    pl.kernel,
    out_type=jax.ShapeDtypeStruct((N_NODES_C * EMB_DIM,), jnp.float32),
    mesh=_mesh,
    compiler_params=_params,
    scratch_types=[
        pltpu.VMEM((MAX_DEGREE * EMB_DIM,), jnp.float32),
        pltpu.VMEM((NT,), jnp.int32),
        pltpu.VMEM((NT,), jnp.int32),
        pltpu.VMEM((NT,), jnp.int32),
        pltpu.VMEM((NT,), jnp.int32),
        pltpu.VMEM((NT,), jnp.int32),
        pltpu.VMEM((NT * EMB_DIM,), jnp.float32),
        pltpu.SemaphoreType.DMA,
        pltpu.SemaphoreType.DMA,
        pltpu.SemaphoreType.DMA,
        pltpu.SemaphoreType.DMA,
    ],
)(_lookup_body)


def kernel(edge_index, num_nodes, emb_weight):
    part = _hist_kernel(edge_index)
    out_flat = _lookup_kernel(part, emb_weight.reshape(-1))
    return out_flat.reshape(N_NODES_C, EMB_DIM)


# speedup vs baseline: 1.0990x; 1.0389x over previous
import functools

import jax
import jax.numpy as jnp
from jax import lax
from jax.experimental import pallas as pl
from jax.experimental.pallas import tpu as pltpu
from jax.experimental.pallas import tpu_sc as plsc

N_NODES_C = 100000
MAX_DEGREE = 512
EMB_DIM = 16

NC, NS, L = 2, 16, 16
NW = NC * NS

E_TOTAL = 3200000
EP = E_TOTAL // NW
RW = 100096
CH = 4096
N_FULL = RW // CH
CH_T = RW - N_FULL * CH

NT = 3136
NSEG = NT // L
NPAD = NW * NT
N_LAST = N_NODES_C - (NW - 1) * NT

_mesh = plsc.VectorSubcoreMesh(core_axis_name="c", subcore_axis_name="s")
_params = pltpu.CompilerParams(needs_layout_passes=False)


def _wid():
    return lax.axis_index("s") * NC + lax.axis_index("c")


def _hist_body(edge_ref, part_ref, hist, ebuf0, ebuf1, sem0, sem1):
    wid = _wid()
    wstart = wid * EP
    awin = wstart - lax.rem(wstart, 128)
    lo = wstart - awin

    zeros16 = jnp.zeros((L,), jnp.int32)
    ones16 = jnp.ones((L,), jnp.int32)
    iota16 = lax.iota(jnp.int32, L)

    def start_chunk(k, buf, sem, ncols):
        off = pl.multiple_of(awin + k * CH, 128)
        pltpu.make_async_copy(
            edge_ref.at[:, pl.ds(off, ncols)],
            buf.at[:, pl.ds(0, ncols)], sem).start()

    def wait_chunk(buf, sem, ncols):
        pltpu.make_async_copy(
            edge_ref.at[:, pl.ds(0, ncols)],
            buf.at[:, pl.ds(0, ncols)], sem).wait()

    def do_group(buf, i, mask):
        idx = buf[0, pl.ds(i * L, L)]
        plsc.addupdate_scatter(hist, [idx], ones16, mask=mask)

    def do_chunk(buf):
        @plsc.parallel_loop(0, CH // L, 1, unroll=8)
        def _(i):
            do_group(buf, i, None)

    def do_chunk_masked_lo(buf):
        @plsc.parallel_loop(0, CH // L, 1, unroll=8)
        def _(i):
            mask = (i * L + iota16) >= lo
            do_group(buf, i, mask)

    def do_chunk_masked_hi(buf):
        hi = lo + (EP - N_FULL * CH)
        @plsc.parallel_loop(0, CH_T // L, 1, unroll=8)
        def _(i):
            mask = (i * L + iota16) < hi
            do_group(buf, i, mask)

    start_chunk(0, ebuf0, sem0, CH)
    start_chunk(1, ebuf1, sem1, CH)

    @plsc.parallel_loop(0, NPAD // L, 1, unroll=16)
    def _(i):
        hist[pl.ds(i * L, L)] = zeros16

    wait_chunk(ebuf0, sem0, CH)
    do_chunk_masked_lo(ebuf0)
    start_chunk(2, ebuf0, sem0, CH)

    def pair_body(j, carry):
        k1 = 2 * j + 1
        wait_chunk(ebuf1, sem1, CH)
        do_chunk(ebuf1)
        start_chunk(k1 + 2, ebuf1, sem1, CH)
        wait_chunk(ebuf0, sem0, CH)
        do_chunk(ebuf0)
        @pl.when(k1 + 3 < N_FULL)
        def _():
            start_chunk(k1 + 3, ebuf0, sem0, CH)
        return carry

    lax.fori_loop(0, (N_FULL - 2) // 2, pair_body, 0)
    wait_chunk(ebuf1, sem1, CH)
    do_chunk(ebuf1)
    start_chunk(N_FULL, ebuf0, sem0, CH_T)
    wait_chunk(ebuf0, sem0, CH_T)
    do_chunk_masked_hi(ebuf0)

    pltpu.sync_copy(hist, part_ref.at[pl.ds(wid * NPAD, NPAD)])


_hist_kernel = functools.partial(
    pl.kernel,
    out_type=jax.ShapeDtypeStruct((NW * NPAD,), jnp.int32),
    mesh=_mesh,
    compiler_params=_params,
    scratch_types=[
        pltpu.VMEM((NPAD,), jnp.int32),
        pltpu.VMEM((2, CH), jnp.int32),
        pltpu.VMEM((2, CH), jnp.int32),
        pltpu.SemaphoreType.DMA,
        pltpu.SemaphoreType.DMA,
    ],
)(_hist_body)


def _lookup_body(part_ref, emb_ref, out_ref, emb_v, acc,
                 pbuf0, pbuf1, pbuf2, pbuf3, outbuf,
                 sem0, sem1, sem2, sem3):
    wid = _wid()
    base = wid * NT
    pbufs = (pbuf0, pbuf1, pbuf2, pbuf3)
    sems = (sem0, sem1, sem2, sem3)

    def start_row(r, buf, sem):
        pltpu.make_async_copy(
            part_ref.at[pl.ds(r * NPAD + base, NT)], buf, sem).start()

    def wait_row(buf, sem):
        pltpu.make_async_copy(part_ref.at[pl.ds(0, NT)], buf, sem).wait()

    def addrow(buf):
        @plsc.parallel_loop(0, NSEG, 1, unroll=8)
        def _(s):
            sl = pl.ds(s * L, L)
            acc[sl] = acc[sl] + buf[sl]

    for b in range(4):
        start_row(1 + b, pbufs[b], sems[b])
    pltpu.sync_copy(emb_ref, emb_v)
    pltpu.sync_copy(part_ref.at[pl.ds(base, NT)], acc)

    def ring_body(j, carry):
        for b in range(4):
            r = 4 * j + 1 + b
            wait_row(pbufs[b], sems[b])
            addrow(pbufs[b])
            @pl.when(r + 4 < NW)
            def _():
                start_row(r + 4, pbufs[b], sems[b])
        return carry

    lax.fori_loop(0, 7, ring_body, 0)
    for b in range(3):
        wait_row(pbufs[b], sems[b])
        addrow(pbufs[b])

    iota16 = lax.iota(jnp.int32, L)
    colvs = [jnp.full((L,), c, jnp.int32) for c in range(EMB_DIM)]

    @plsc.parallel_loop(0, NSEG, 1, unroll=2)
    def _(s):
        d = acc[pl.ds(s * L, L)]
        dc = jnp.minimum(jnp.maximum(d, 1), MAX_DEGREE) - 1
        src_base = dc * EMB_DIM
        rowi = s * L + iota16
        for c in range(EMB_DIM):
            vals = plsc.load_gather(emb_v, [src_base + c])
            plsc.store_scatter(outbuf, [rowi, colvs[c]], vals)

    @pl.when(wid < NW - 1)
    def _():
        pltpu.sync_copy(outbuf, out_ref.at[pl.ds(base, NT), :])

    @pl.when(wid == NW - 1)
    def _():
        pltpu.sync_copy(outbuf.at[pl.ds(0, N_LAST), :],
                        out_ref.at[pl.ds(base, N_LAST), :])


_lookup_kernel = functools.partial(
    pl.kernel,
    out_type=jax.ShapeDtypeStruct((N_NODES_C, EMB_DIM), jnp.float32),
    mesh=_mesh,
    compiler_params=pltpu.CompilerParams(needs_layout_passes=False,
                                         use_tc_tiling_on_sc=False),
    scratch_types=[
        pltpu.VMEM((MAX_DEGREE * EMB_DIM,), jnp.float32),
        pltpu.VMEM((NT,), jnp.int32),
        pltpu.VMEM((NT,), jnp.int32),
        pltpu.VMEM((NT,), jnp.int32),
        pltpu.VMEM((NT,), jnp.int32),
        pltpu.VMEM((NT,), jnp.int32),
        pltpu.VMEM((NT, EMB_DIM), jnp.float32),
        pltpu.SemaphoreType.DMA,
        pltpu.SemaphoreType.DMA,
        pltpu.SemaphoreType.DMA,
        pltpu.SemaphoreType.DMA,
    ],
)(_lookup_body)


def kernel(edge_index, num_nodes, emb_weight):
    part = _hist_kernel(edge_index)
    return _lookup_kernel(part, emb_weight.reshape(-1))
